# baseline (speedup 1.0000x reference)
import functools

import jax
import jax.numpy as jnp
from jax import lax
from jax.experimental import pallas as pl
from jax.experimental.pallas import tpu as pltpu
from jax.experimental.pallas import tpu_sc as plsc

_B, _M, _T, _U, _W, _H = 64, 32, 2, 16, 24, 24
_C = 24
_CELLS = _W * _H
_NB = 4
_CCEL = _H
_CROWS = _CCEL * _C
_HROWS = _CROWS // 2


def _sc_body(tt_hbm, ints_hbm, flts_hbm, ttab_hbm, utab_hbm, out_hbm,
             tt_v, ints_v, flts_v, ttab_v, utab_v, buf_v,
             ttsem, osemA, osemB):
    m = lax.axis_index("c") * 16 + lax.axis_index("s")
    out3 = out_hbm.reshape(_M, _CELLS * _C, _B)

    pltpu.sync_copy(ttab_hbm, ttab_v)
    pltpu.sync_copy(utab_hbm, utab_v)
    pltpu.sync_copy(ints_hbm.at[m], ints_v)
    pltpu.sync_copy(flts_hbm.at[m], flts_v)
    pltpu.async_copy(tt_hbm.at[m, pl.ds(0, _CCEL), :], tt_v.at[0], ttsem)

    lanes = lax.iota(jnp.int32, 16)
    zero_v = jnp.zeros((16,), jnp.float32)

    def half_out(ci, h):
        return out3.at[m, pl.ds(ci * _CROWS + h * _HROWS, _HROWS), :]

    def chunk_body(ci, carry0):
        par = lax.rem(ci, 2)
        w0 = ci

        pltpu.make_async_copy(tt_hbm.at[m, pl.ds(0, _CCEL), :],
                              tt_v.at[0], ttsem).wait()

        @pl.when(ci < _W - 1)
        def _():
            pltpu.async_copy(tt_hbm.at[m, pl.ds((ci + 1) * _CCEL, _CCEL), :],
                             tt_v.at[1 - par], ttsem)

        @pl.when(ci > 0)
        def _():
            pltpu.make_async_copy(buf_v.at[pl.ds(0, _HROWS)],
                                  half_out(ci, 0), osemA).wait()
            pltpu.make_async_copy(buf_v.at[pl.ds(_HROWS, _HROWS)],
                                  half_out(ci, 1), osemB).wait()

        def zero_body(lc, carry1):
            row0 = lc * _C
            for c in range(4, 23):
                for cb in range(_NB):
                    buf_v[row0 + c, pl.ds(cb * 16, 16)] = zero_v
            return carry1
        lax.fori_loop(0, _CCEL, zero_body, 0)

        def bblk_scatter(bb, carry1):
            boff = bb * 16
            for t in range(_T):
                for u in range(_U):
                    xv = ints_v[t * 16 + u, pl.ds(boff, 16)]
                    inb = xv == w0
                    hit = plsc.all_reduce_population_count(inb)

                    @pl.when(hit[0] > 0)
                    def _(t=t, u=u, boff=boff, inb=inb):
                        yv = ints_v[32 + t * 16 + u, pl.ds(boff, 16)]
                        mv = flts_v[t * 16 + u, pl.ds(boff, 16)]
                        meff = jnp.where(inb, mv, 0.0)
                        rowv = yv * _C
                        urow = utab_v[u]
                        for e in range(9):
                            c = 4 + t if e == 0 else 5 + 8 * t + e
                            plsc.addupdate_scatter(
                                buf_v, [rowv + c, boff + lanes],
                                meff * urow[e], mask=inb)
            for r in range(6):
                xv = ints_v[64 + r, pl.ds(boff, 16)]
                yv = ints_v[70 + r, pl.ds(boff, 16)]
                mv = flts_v[32 + r, pl.ds(boff, 16)]
                inb = xv == w0
                meff = jnp.where(inb, mv, 0.0)
                rowv = yv * _C + 22
                plsc.addupdate_scatter(buf_v, [rowv, boff + lanes],
                                       meff, mask=inb)
            return carry1
        lax.fori_loop(0, _NB, bblk_scatter, 0)

        nrvs = [flts_v[38, pl.ds(bb * 16, 16)] for bb in range(_NB)]

        def dense(lc0):
            def cell_body(lc, carry2):
                row0 = lc * _C
                for bb in range(_NB):
                    boff = bb * 16
                    idx4 = tt_v[par, lc, pl.ds(boff, 16)] * 4
                    for e in range(4):
                        buf_v[row0 + e, pl.ds(boff, 16)] = (
                            plsc.load_gather(ttab_v, [idx4 + e]))
                    buf_v[row0 + 23, pl.ds(boff, 16)] = nrvs[bb]
                return carry2
            lax.fori_loop(lc0, lc0 + _CCEL // 2, cell_body, 0)

        dense(0)
        pltpu.async_copy(buf_v.at[pl.ds(0, _HROWS)], half_out(ci, 0), osemA)
        dense(_CCEL // 2)
        pltpu.async_copy(buf_v.at[pl.ds(_HROWS, _HROWS)], half_out(ci, 1),
                         osemB)
        return carry0

    lax.fori_loop(0, _W, chunk_body, 0)
    pltpu.make_async_copy(buf_v.at[pl.ds(0, _HROWS)],
                          half_out(0, 0), osemA).wait()
    pltpu.make_async_copy(buf_v.at[pl.ds(_HROWS, _HROWS)],
                          half_out(0, 1), osemB).wait()


def _sc_call(tt, ints, flts, ttab, utab):
    mesh = plsc.VectorSubcoreMesh(core_axis_name="c", subcore_axis_name="s")
    return pl.kernel(
        _sc_body,
        out_type=jax.ShapeDtypeStruct((_M, _W, _H, _C, _B), jnp.float32),
        mesh=mesh,
        compiler_params=pltpu.CompilerParams(needs_layout_passes=False),
        scratch_types=[
            pltpu.VMEM((2, _CCEL, _B), jnp.int32),
            pltpu.VMEM((76, _B), jnp.int32),
            pltpu.VMEM((40, _B), jnp.float32),
            pltpu.VMEM((16,), jnp.float32),
            pltpu.VMEM((16, 16), jnp.float32),
            pltpu.VMEM((_CROWS, _B), jnp.float32),
            pltpu.SemaphoreType.DMA,
            pltpu.SemaphoreType.DMA,
            pltpu.SemaphoreType.DMA,
        ],
    )(tt, ints, flts, ttab, utab)


def kernel(position, units_mask, relic_positions, relic_mask, tile_type,
           normalized_reward, tile_table, unit_table):
    f32 = jnp.float32
    i32 = jnp.int32
    tt = tile_type.astype(i32).transpose(1, 2, 3, 0).reshape(_M, _CELLS, _B)
    x = position[..., 0].transpose(1, 2, 3, 0).reshape(_M, _T * _U, _B)
    y = position[..., 1].transpose(1, 2, 3, 0).reshape(_M, _T * _U, _B)
    rx = relic_positions[..., 0].transpose(1, 2, 0)
    ry = relic_positions[..., 1].transpose(1, 2, 0)
    ints = jnp.concatenate([x, y, rx, ry], axis=1)
    um = units_mask.astype(f32).transpose(1, 2, 3, 0).reshape(_M, _T * _U, _B)
    rm = relic_mask.astype(f32).transpose(1, 2, 0)
    nr = normalized_reward.astype(f32).T[:, None, :]
    flts = jnp.concatenate([um, rm, nr, jnp.zeros((_M, 1, _B), f32)], axis=1)
    ttab = jnp.concatenate([tile_table.reshape(12).astype(f32),
                            jnp.zeros((4,), f32)])
    utab = jnp.concatenate(
        [jnp.full((_U, 1), 1.0 / _U, f32), unit_table.astype(f32),
         jnp.zeros((_U, 7), f32)], axis=1)
    out = _sc_call(tt, ints, flts, ttab, utab)
    return out.transpose(4, 0, 1, 2, 3)


# speedup vs baseline: 1.3545x; 1.3545x over previous
import functools

import jax
import jax.numpy as jnp
from jax import lax
from jax.experimental import pallas as pl
from jax.experimental.pallas import tpu as pltpu
from jax.experimental.pallas import tpu_sc as plsc

_B, _M, _T, _U, _W, _H = 64, 32, 2, 16, 24, 24
_C = 24
_CELLS = _W * _H
_NB = 4
_CCEL = _H
_CROWS = _CCEL * _C
_HROWS = _CROWS // 2


def _sc_body(tt_hbm, ints_hbm, flts_hbm, ttab_hbm, utab_hbm, out_hbm,
             tt_v, ints_v, flts_v, ttab_v, utab_v, buf_v,
             ttsem, osemA, osemB):
    m = lax.axis_index("c") * 16 + lax.axis_index("s")
    out3 = out_hbm.reshape(_M, _CELLS * _C, _B)

    pltpu.sync_copy(ttab_hbm, ttab_v)
    pltpu.sync_copy(utab_hbm, utab_v)
    pltpu.sync_copy(ints_hbm.at[m], ints_v)
    pltpu.sync_copy(flts_hbm.at[m], flts_v)
    pltpu.async_copy(tt_hbm.at[m, pl.ds(0, _CCEL), :], tt_v.at[0], ttsem)

    lanes = lax.iota(jnp.int32, 16)
    zero_v = jnp.zeros((16,), jnp.float32)

    def half_out(ci, h):
        return out3.at[m, pl.ds(ci * _CROWS + h * _HROWS, _HROWS), :]

    def chunk_body(ci, carry0):
        par = lax.rem(ci, 2)
        w0 = ci

        pltpu.make_async_copy(tt_hbm.at[m, pl.ds(0, _CCEL), :],
                              tt_v.at[0], ttsem).wait()

        @pl.when(ci < _W - 1)
        def _():
            pltpu.async_copy(tt_hbm.at[m, pl.ds((ci + 1) * _CCEL, _CCEL), :],
                             tt_v.at[1 - par], ttsem)

        @pl.when(ci > 0)
        def _():
            pltpu.make_async_copy(buf_v.at[pl.ds(0, _HROWS)],
                                  half_out(ci, 0), osemA).wait()
            pltpu.make_async_copy(buf_v.at[pl.ds(_HROWS, _HROWS)],
                                  half_out(ci, 1), osemB).wait()

        def zero_body(lc, carry1):
            row0 = lc * _C
            for c in range(4, 23):
                for cb in range(_NB):
                    buf_v[row0 + c, pl.ds(cb * 16, 16)] = zero_v
            return carry1
        lax.fori_loop(0, _CCEL, zero_body, 0)

        def bblk_scatter(bb, carry1):
            boff = bb * 16
            for t in range(_T):
                for u in range(_U):
                    xv = ints_v[t * 16 + u, pl.ds(boff, 16)]
                    yv = ints_v[32 + t * 16 + u, pl.ds(boff, 16)]
                    mv = flts_v[t * 16 + u, pl.ds(boff, 16)]
                    inb = xv == w0
                    meff = jnp.where(inb, mv, 0.0)
                    rowv = yv * _C
                    urow = utab_v[u]
                    for e in range(9):
                        c = 4 + t if e == 0 else 5 + 8 * t + e
                        plsc.addupdate_scatter(
                            buf_v, [rowv + c, boff + lanes],
                            meff * urow[e], mask=inb)
            for r in range(6):
                xv = ints_v[64 + r, pl.ds(boff, 16)]
                yv = ints_v[70 + r, pl.ds(boff, 16)]
                mv = flts_v[32 + r, pl.ds(boff, 16)]
                inb = xv == w0
                meff = jnp.where(inb, mv, 0.0)
                rowv = yv * _C + 22
                plsc.addupdate_scatter(buf_v, [rowv, boff + lanes],
                                       meff, mask=inb)
            return carry1
        lax.fori_loop(0, _NB, bblk_scatter, 0)

        nrvs = [flts_v[38, pl.ds(bb * 16, 16)] for bb in range(_NB)]

        def dense(lc0):
            def cell_body(lc, carry2):
                row0 = lc * _C
                vals = []
                for bb in range(_NB):
                    idx4 = tt_v[par, lc, pl.ds(bb * 16, 16)] * 4
                    vals.append([plsc.load_gather(ttab_v, [idx4 + e])
                                 for e in range(4)])
                for bb in range(_NB):
                    boff = bb * 16
                    for e in range(4):
                        buf_v[row0 + e, pl.ds(boff, 16)] = vals[bb][e]
                    buf_v[row0 + 23, pl.ds(boff, 16)] = nrvs[bb]
                return carry2
            lax.fori_loop(lc0, lc0 + _CCEL // 2, cell_body, 0, unroll=2)

        dense(0)
        pltpu.async_copy(buf_v.at[pl.ds(0, _HROWS)], half_out(ci, 0), osemA)
        dense(_CCEL // 2)
        pltpu.async_copy(buf_v.at[pl.ds(_HROWS, _HROWS)], half_out(ci, 1),
                         osemB)
        return carry0

    lax.fori_loop(0, _W, chunk_body, 0)
    pltpu.make_async_copy(buf_v.at[pl.ds(0, _HROWS)],
                          half_out(0, 0), osemA).wait()
    pltpu.make_async_copy(buf_v.at[pl.ds(_HROWS, _HROWS)],
                          half_out(0, 1), osemB).wait()


def _sc_call(tt, ints, flts, ttab, utab):
    mesh = plsc.VectorSubcoreMesh(core_axis_name="c", subcore_axis_name="s")
    return pl.kernel(
        _sc_body,
        out_type=jax.ShapeDtypeStruct((_M, _W, _H, _C, _B), jnp.float32),
        mesh=mesh,
        compiler_params=pltpu.CompilerParams(needs_layout_passes=False),
        scratch_types=[
            pltpu.VMEM((2, _CCEL, _B), jnp.int32),
            pltpu.VMEM((76, _B), jnp.int32),
            pltpu.VMEM((40, _B), jnp.float32),
            pltpu.VMEM((16,), jnp.float32),
            pltpu.VMEM((16, 16), jnp.float32),
            pltpu.VMEM((_CROWS, _B), jnp.float32),
            pltpu.SemaphoreType.DMA,
            pltpu.SemaphoreType.DMA,
            pltpu.SemaphoreType.DMA,
        ],
    )(tt, ints, flts, ttab, utab)


def kernel(position, units_mask, relic_positions, relic_mask, tile_type,
           normalized_reward, tile_table, unit_table):
    f32 = jnp.float32
    i32 = jnp.int32
    tt = tile_type.astype(i32).transpose(1, 2, 3, 0).reshape(_M, _CELLS, _B)
    x = position[..., 0].transpose(1, 2, 3, 0).reshape(_M, _T * _U, _B)
    y = position[..., 1].transpose(1, 2, 3, 0).reshape(_M, _T * _U, _B)
    rx = relic_positions[..., 0].transpose(1, 2, 0)
    ry = relic_positions[..., 1].transpose(1, 2, 0)
    ints = jnp.concatenate([x, y, rx, ry], axis=1)
    um = units_mask.astype(f32).transpose(1, 2, 3, 0).reshape(_M, _T * _U, _B)
    rm = relic_mask.astype(f32).transpose(1, 2, 0)
    nr = normalized_reward.astype(f32).T[:, None, :]
    flts = jnp.concatenate([um, rm, nr, jnp.zeros((_M, 1, _B), f32)], axis=1)
    ttab = jnp.concatenate([tile_table.reshape(12).astype(f32),
                            jnp.zeros((4,), f32)])
    utab = jnp.concatenate(
        [jnp.full((_U, 1), 1.0 / _U, f32), unit_table.astype(f32),
         jnp.zeros((_U, 7), f32)], axis=1)
    out = _sc_call(tt, ints, flts, ttab, utab)
    return out.transpose(4, 0, 1, 2, 3)
